# baseline (speedup 1.0000x reference)
import functools

import jax
import jax.numpy as jnp
from jax import lax
from jax.experimental import pallas as pl
from jax.experimental.pallas import tpu as pltpu
from jax.experimental.pallas import tpu_sc as plsc

N = 10000
NP = 10240
E = 320000
D = 128
NC, NS = 2, 16
NW = NC * NS
CK = 128
CHUNKS = 79
EP = NW * CHUNKS * CK
ROWS_PER_TILE = NP // NS
EPS = 1e-5

_mesh = plsc.VectorSubcoreMesh(core_axis_name="c", subcore_axis_name="s",
                               num_cores=NC, num_subcores=NS)


@functools.partial(
    pl.kernel,
    out_type=jax.ShapeDtypeStruct((NC, NP, D), jnp.float32),
    mesh=_mesh,
    scratch_types=[
        pltpu.VMEM((40, CK), jnp.int32),
        pltpu.VMEM((40, CK), jnp.int32),
        pltpu.VMEM((2, CK, D), jnp.float32),
        pltpu.VMEM_SHARED((NP, D), jnp.float32),
        pltpu.SemaphoreType.DMA,
        pltpu.SemaphoreType.DMA,
        pltpu.SemaphoreType.DMA,
        pltpu.SemaphoreType.DMA,
    ],
)
def _sc_scatter(g_hbm, src_hbm, dst_hbm, zeros_hbm, out_hbm,
                idx_s, idx_d, rows, acc, gsem0, gsem1, ssem0, ssem1):
    c = lax.axis_index("c")
    s = lax.axis_index("s")
    w = c * NS + s
    pltpu.sync_copy(zeros_hbm, acc.at[pl.ds(s * ROWS_PER_TILE, ROWS_PER_TILE)])
    plsc.subcore_barrier()

    for base, n in ((0, 40), (40, CHUNKS - 40)):
        pltpu.sync_copy(src_hbm.at[w, pl.ds(base, n)], idx_s.at[pl.ds(0, n)])
        pltpu.sync_copy(dst_hbm.at[w, pl.ds(base, n)], idx_d.at[pl.ds(0, n)])
        pltpu.async_copy(g_hbm.at[idx_s.at[0]], rows.at[0], gsem0)

        def body(i, carry):
            j0 = i * 2
            pltpu.async_copy(g_hbm.at[idx_s.at[j0 + 1]], rows.at[1], gsem1)
            pltpu.make_async_copy(g_hbm.at[idx_s.at[j0]], rows.at[0], gsem0).wait()
            pltpu.sync_copy(rows.at[0], acc.at[idx_d.at[j0]], add=True)

            @pl.when(j0 + 2 < n)
            def _():
                pltpu.async_copy(g_hbm.at[idx_s.at[j0 + 2]], rows.at[0], gsem0)

            pltpu.make_async_copy(g_hbm.at[idx_s.at[j0 + 1]], rows.at[1], gsem1).wait()
            pltpu.sync_copy(rows.at[1], acc.at[idx_d.at[j0 + 1]], add=True)
            return carry

        lax.fori_loop(0, n // 2, body, 0)
        if n % 2 == 1:
            pltpu.make_async_copy(g_hbm.at[idx_s.at[n - 1]], rows.at[0], gsem0).wait()
            pltpu.sync_copy(rows.at[0], acc.at[idx_d.at[n - 1]], add=True)
    plsc.subcore_barrier()
    pltpu.sync_copy(acc.at[pl.ds(s * ROWS_PER_TILE, ROWS_PER_TILE)],
                    out_hbm.at[c, pl.ds(s * ROWS_PER_TILE, ROWS_PER_TILE)])


@functools.partial(
    pl.kernel,
    out_type=jax.ShapeDtypeStruct((NC, NP, D), jnp.float32),
    mesh=_mesh,
    scratch_types=[
        pltpu.VMEM((CHUNKS, CK), jnp.int32),
        pltpu.VMEM((CK, D), jnp.float32),
        pltpu.VMEM_SHARED((NP, D), jnp.float32),
    ],
)
def _sc_degree(dst_hbm, zeros_hbm, ones_hbm, out_hbm, idx_d, ones_v, acc):
    c = lax.axis_index("c")
    s = lax.axis_index("s")
    w = c * NS + s
    pltpu.sync_copy(zeros_hbm, acc.at[pl.ds(s * ROWS_PER_TILE, ROWS_PER_TILE)])
    pltpu.sync_copy(ones_hbm, ones_v)
    pltpu.sync_copy(dst_hbm.at[w], idx_d)
    plsc.subcore_barrier()

    def body(j, carry):
        pltpu.sync_copy(ones_v, acc.at[idx_d.at[j]], add=True)
        return carry

    lax.fori_loop(0, CHUNKS, body, 0)
    plsc.subcore_barrier()
    pltpu.sync_copy(acc.at[pl.ds(s * ROWS_PER_TILE, ROWS_PER_TILE)],
                    out_hbm.at[c, pl.ds(s * ROWS_PER_TILE, ROWS_PER_TILE)])


def _tx_body(x_ref, w_ref, xw_ref):
    xw_ref[...] = jnp.dot(x_ref[...], w_ref[...],
                          preferred_element_type=jnp.float32)


def _t0_body(degp_ref, xw_ref, dinv_ref, g_ref):
    deg = degp_ref[0, :, 0:1] + degp_ref[1, :, 0:1] + 1.0
    row = lax.broadcasted_iota(jnp.int32, (NP, 1), 0)
    dinv = jnp.where(row < N, lax.rsqrt(deg), 0.0)
    dinv_ref[...] = dinv
    g_ref[...] = dinv * xw_ref[...]


def _t1_body(p_ref, g_ref, dinv_ref, b_ref, wn_ref, h_ref, gn_ref):
    dinv = dinv_ref[...]
    mask = (dinv > 0.0).astype(jnp.float32)
    z = (dinv * (p_ref[0] + p_ref[1] + g_ref[...]) + b_ref[...]) * mask
    h = jnp.maximum(z, 0.0)
    h_ref[...] = h
    gn_ref[...] = dinv * jnp.dot(h, wn_ref[...],
                                 preferred_element_type=jnp.float32)


def _tmid_body(p_ref, g_ref, hin_ref, dinv_ref, b_ref, gam_ref, bet_ref,
               wn_ref, h_ref, gn_ref):
    dinv = dinv_ref[...]
    mask = (dinv > 0.0).astype(jnp.float32)
    z = (dinv * (p_ref[0] + p_ref[1] + g_ref[...]) + b_ref[...]) * mask
    mu = jnp.sum(z, axis=0, keepdims=True) * (1.0 / N)
    d0 = (z - mu) * mask
    var = jnp.sum(d0 * d0, axis=0, keepdims=True) * (1.0 / N)
    bn = (z - mu) * lax.rsqrt(var + EPS) * gam_ref[...] + bet_ref[...]
    h = hin_ref[...] + jnp.maximum(bn, 0.0)
    h_ref[...] = h
    gn_ref[...] = dinv * jnp.dot(h, wn_ref[...],
                                 preferred_element_type=jnp.float32)


def _tlast_body(p_ref, g_ref, hin_ref, dinv_ref, b_ref, gam_ref, bet_ref,
                batch_ref, wm0_ref, bm0_ref, wm1_ref, bm1_ref,
                wm2_ref, bm2_ref, y_ref):
    dinv = dinv_ref[...]
    mask = (dinv > 0.0).astype(jnp.float32)
    z = (dinv * (p_ref[0] + p_ref[1] + g_ref[...]) + b_ref[...]) * mask
    mu = jnp.sum(z, axis=0, keepdims=True) * (1.0 / N)
    d0 = (z - mu) * mask
    var = jnp.sum(d0 * d0, axis=0, keepdims=True) * (1.0 / N)
    bn = (z - mu) * lax.rsqrt(var + EPS) * gam_ref[...] + bet_ref[...]
    h = hin_ref[...] + jnp.maximum(bn, 0.0)
    b = batch_ref[...]
    gids = lax.broadcasted_iota(jnp.int32, (NP, 128), 1)
    onehot = (b == gids).astype(jnp.float32)
    dn = (((0,), (0,)), ((), ()))
    sums = lax.dot_general(onehot, h, dn,
                           preferred_element_type=jnp.float32)
    cnt = lax.dot_general(onehot, jnp.ones((NP, 1), jnp.float32), dn,
                          preferred_element_type=jnp.float32)
    pooled = sums / jnp.maximum(cnt, 1.0)
    y = jnp.maximum(jnp.dot(pooled, wm0_ref[...],
                            preferred_element_type=jnp.float32) + bm0_ref[...], 0.0)
    y = jnp.maximum(jnp.dot(y, wm1_ref[...],
                            preferred_element_type=jnp.float32) + bm1_ref[...], 0.0)
    y_ref[...] = jnp.dot(y, wm2_ref[...],
                         preferred_element_type=jnp.float32) + bm2_ref[...]


def _sds(shape):
    return jax.ShapeDtypeStruct(shape, jnp.float32)


def kernel(x, edge_index, batch, W0, b0, W1, b1, W2, b2, W3, b3,
           gamma, beta, Wm0, bm0, Wm1, bm1, Wm2, bm2):
    f32 = jnp.float32
    src = edge_index[0].astype(jnp.int32)
    dst = edge_index[1].astype(jnp.int32)
    pad_idx = N + (jnp.arange(EP - E, dtype=jnp.int32) % (NP - N))
    src_p = jnp.concatenate([src, pad_idx]).reshape(NW, CHUNKS, CK)
    dst_p = jnp.concatenate([dst, pad_idx]).reshape(NW, CHUNKS, CK)
    x_p = jnp.concatenate([x.astype(f32), jnp.zeros((NP - N, D), f32)])
    batch_p = jnp.concatenate(
        [batch.astype(jnp.int32), jnp.full((NP - N,), 1 << 20, jnp.int32)]
    ).reshape(NP, 1)
    zrow = jnp.zeros((ROWS_PER_TILE, D), f32)
    onesrow = jnp.ones((CK, D), f32)
    b0r, b1r, b2r, b3r = (v.reshape(1, D) for v in (b0, b1, b2, b3))
    gam, bet = gamma.reshape(1, D), beta.reshape(1, D)

    degp = _sc_degree(dst_p, zrow, onesrow)
    xw = pl.pallas_call(_tx_body, out_shape=_sds((NP, D)))(x_p, W0)

    dinv, g = pl.pallas_call(
        _t0_body, out_shape=[_sds((NP, 1)), _sds((NP, D))])(degp, xw)

    parts = _sc_scatter(g, src_p, dst_p, zrow)
    h, g = pl.pallas_call(
        _t1_body, out_shape=[_sds((NP, D)), _sds((NP, D))])(
            parts, g, dinv, b0r, W1)

    for bb, wn in ((b1r, W2), (b2r, W3)):
        parts = _sc_scatter(g, src_p, dst_p, zrow)
        h, g = pl.pallas_call(
            _tmid_body, out_shape=[_sds((NP, D)), _sds((NP, D))])(
                parts, g, h, dinv, bb, gam, bet, wn)

    parts = _sc_scatter(g, src_p, dst_p, zrow)
    y = pl.pallas_call(_tlast_body, out_shape=_sds((128, 10)))(
        parts, g, h, dinv, b3r, gam, bet, batch_p, Wm0, bm0.reshape(1, -1),
        Wm1, bm1.reshape(1, -1), Wm2, bm2.reshape(1, -1))
    return y


# speedup vs baseline: 1.0018x; 1.0018x over previous
import functools

import jax
import jax.numpy as jnp
from jax import lax
from jax.experimental import pallas as pl
from jax.experimental.pallas import tpu as pltpu
from jax.experimental.pallas import tpu_sc as plsc

N = 10000
NP = 10240
E = 320000
D = 128
NC, NS = 2, 16
NW = NC * NS
CK = 128
CHUNKS = 79
EP = NW * CHUNKS * CK
ROWS_PER_TILE = NP // NS
EPS = 1e-5

_mesh = plsc.VectorSubcoreMesh(core_axis_name="c", subcore_axis_name="s",
                               num_cores=NC, num_subcores=NS)


@functools.partial(
    pl.kernel,
    out_type=jax.ShapeDtypeStruct((NC, NP, D), jnp.float32),
    mesh=_mesh,
    scratch_types=[
        pltpu.VMEM((40, CK), jnp.int32),
        pltpu.VMEM((40, CK), jnp.int32),
        pltpu.VMEM((2, CK, D), jnp.float32),
        pltpu.VMEM_SHARED((NP, D), jnp.float32),
        pltpu.SemaphoreType.DMA,
        pltpu.SemaphoreType.DMA,
        pltpu.SemaphoreType.DMA,
        pltpu.SemaphoreType.DMA,
    ],
)
def _sc_scatter(g_hbm, src_hbm, dst_hbm, zeros_hbm, out_hbm,
                idx_s, idx_d, rows, acc, gsem0, gsem1, ssem0, ssem1):
    c = lax.axis_index("c")
    s = lax.axis_index("s")
    w = c * NS + s
    pltpu.sync_copy(zeros_hbm, acc.at[pl.ds(s * ROWS_PER_TILE, ROWS_PER_TILE)])
    plsc.subcore_barrier()

    for base, n in ((0, 40), (40, CHUNKS - 40)):
        pltpu.sync_copy(src_hbm.at[w, pl.ds(base, n)], idx_s.at[pl.ds(0, n)])
        pltpu.sync_copy(dst_hbm.at[w, pl.ds(base, n)], idx_d.at[pl.ds(0, n)])
        pltpu.async_copy(g_hbm.at[idx_s.at[0]], rows.at[0], gsem0)

        def body(i, carry):
            j0 = i * 2
            pltpu.async_copy(g_hbm.at[idx_s.at[j0 + 1]], rows.at[1], gsem1)
            pltpu.make_async_copy(g_hbm.at[idx_s.at[j0]], rows.at[0], gsem0).wait()
            pltpu.sync_copy(rows.at[0], acc.at[idx_d.at[j0]], add=True)

            @pl.when(j0 + 2 < n)
            def _():
                pltpu.async_copy(g_hbm.at[idx_s.at[j0 + 2]], rows.at[0], gsem0)

            pltpu.make_async_copy(g_hbm.at[idx_s.at[j0 + 1]], rows.at[1], gsem1).wait()
            pltpu.sync_copy(rows.at[1], acc.at[idx_d.at[j0 + 1]], add=True)
            return carry

        lax.fori_loop(0, n // 2, body, 0)
        if n % 2 == 1:
            pltpu.make_async_copy(g_hbm.at[idx_s.at[n - 1]], rows.at[0], gsem0).wait()
            pltpu.sync_copy(rows.at[0], acc.at[idx_d.at[n - 1]], add=True)
    plsc.subcore_barrier()
    pltpu.sync_copy(acc.at[pl.ds(s * ROWS_PER_TILE, ROWS_PER_TILE)],
                    out_hbm.at[c, pl.ds(s * ROWS_PER_TILE, ROWS_PER_TILE)])


@functools.partial(
    pl.kernel,
    out_type=jax.ShapeDtypeStruct((NC, NP, D), jnp.float32),
    mesh=_mesh,
    scratch_types=[
        pltpu.VMEM((CHUNKS, CK), jnp.int32),
        pltpu.VMEM((CK, D), jnp.float32),
        pltpu.VMEM_SHARED((NP, D), jnp.float32),
    ],
)
def _sc_degree(dst_hbm, zeros_hbm, ones_hbm, out_hbm, idx_d, ones_v, acc):
    c = lax.axis_index("c")
    s = lax.axis_index("s")
    w = c * NS + s
    pltpu.sync_copy(zeros_hbm, acc.at[pl.ds(s * ROWS_PER_TILE, ROWS_PER_TILE)])
    pltpu.sync_copy(ones_hbm, ones_v)
    pltpu.sync_copy(dst_hbm.at[w], idx_d)
    plsc.subcore_barrier()

    def body(j, carry):
        pltpu.sync_copy(ones_v, acc.at[idx_d.at[j]], add=True)
        return carry

    lax.fori_loop(0, CHUNKS, body, 0)
    plsc.subcore_barrier()
    pltpu.sync_copy(acc.at[pl.ds(s * ROWS_PER_TILE, ROWS_PER_TILE)],
                    out_hbm.at[c, pl.ds(s * ROWS_PER_TILE, ROWS_PER_TILE)])


def _t0_body(degp_ref, x_ref, w_ref, dinv_ref, g_ref):
    deg = degp_ref[0, :, 0:1] + degp_ref[1, :, 0:1] + 1.0
    row = lax.broadcasted_iota(jnp.int32, (NP, 1), 0)
    dinv = jnp.where(row < N, lax.rsqrt(deg), 0.0)
    dinv_ref[...] = dinv
    g_ref[...] = dinv * jnp.dot(x_ref[...], w_ref[...],
                                preferred_element_type=jnp.float32)


def _t1_body(p_ref, g_ref, dinv_ref, b_ref, wn_ref, h_ref, gn_ref):
    dinv = dinv_ref[...]
    mask = (dinv > 0.0).astype(jnp.float32)
    z = (dinv * (p_ref[0] + p_ref[1] + g_ref[...]) + b_ref[...]) * mask
    h = jnp.maximum(z, 0.0)
    h_ref[...] = h
    gn_ref[...] = dinv * jnp.dot(h, wn_ref[...],
                                 preferred_element_type=jnp.float32)


def _tmid_body(p_ref, g_ref, hin_ref, dinv_ref, b_ref, gam_ref, bet_ref,
               wn_ref, h_ref, gn_ref):
    dinv = dinv_ref[...]
    mask = (dinv > 0.0).astype(jnp.float32)
    z = (dinv * (p_ref[0] + p_ref[1] + g_ref[...]) + b_ref[...]) * mask
    mu = jnp.sum(z, axis=0, keepdims=True) * (1.0 / N)
    d0 = (z - mu) * mask
    var = jnp.sum(d0 * d0, axis=0, keepdims=True) * (1.0 / N)
    bn = (z - mu) * lax.rsqrt(var + EPS) * gam_ref[...] + bet_ref[...]
    h = hin_ref[...] + jnp.maximum(bn, 0.0)
    h_ref[...] = h
    gn_ref[...] = dinv * jnp.dot(h, wn_ref[...],
                                 preferred_element_type=jnp.float32)


def _tlast_body(p_ref, g_ref, hin_ref, dinv_ref, b_ref, gam_ref, bet_ref,
                batch_ref, wm0_ref, bm0_ref, wm1_ref, bm1_ref,
                wm2_ref, bm2_ref, y_ref):
    dinv = dinv_ref[...]
    mask = (dinv > 0.0).astype(jnp.float32)
    z = (dinv * (p_ref[0] + p_ref[1] + g_ref[...]) + b_ref[...]) * mask
    mu = jnp.sum(z, axis=0, keepdims=True) * (1.0 / N)
    d0 = (z - mu) * mask
    var = jnp.sum(d0 * d0, axis=0, keepdims=True) * (1.0 / N)
    bn = (z - mu) * lax.rsqrt(var + EPS) * gam_ref[...] + bet_ref[...]
    h = hin_ref[...] + jnp.maximum(bn, 0.0)
    b = batch_ref[...]
    gids = lax.broadcasted_iota(jnp.int32, (NP, 128), 1)
    onehot = (b == gids).astype(jnp.float32)
    dn = (((0,), (0,)), ((), ()))
    sums = lax.dot_general(onehot, h, dn,
                           preferred_element_type=jnp.float32)
    cnt = lax.dot_general(onehot, jnp.ones((NP, 1), jnp.float32), dn,
                          preferred_element_type=jnp.float32)
    pooled = sums / jnp.maximum(cnt, 1.0)
    y = jnp.maximum(jnp.dot(pooled, wm0_ref[...],
                            preferred_element_type=jnp.float32) + bm0_ref[...], 0.0)
    y = jnp.maximum(jnp.dot(y, wm1_ref[...],
                            preferred_element_type=jnp.float32) + bm1_ref[...], 0.0)
    y_ref[...] = jnp.dot(y, wm2_ref[...],
                         preferred_element_type=jnp.float32) + bm2_ref[...]


def _sds(shape):
    return jax.ShapeDtypeStruct(shape, jnp.float32)


def kernel(x, edge_index, batch, W0, b0, W1, b1, W2, b2, W3, b3,
           gamma, beta, Wm0, bm0, Wm1, bm1, Wm2, bm2):
    f32 = jnp.float32
    src = edge_index[0].astype(jnp.int32)
    dst = edge_index[1].astype(jnp.int32)
    pad_idx = N + (jnp.arange(EP - E, dtype=jnp.int32) % (NP - N))
    src_p = jnp.concatenate([src, pad_idx]).reshape(NW, CHUNKS, CK)
    dst_p = jnp.concatenate([dst, pad_idx]).reshape(NW, CHUNKS, CK)
    x_p = jnp.concatenate([x.astype(f32), jnp.zeros((NP - N, D), f32)])
    batch_p = jnp.concatenate(
        [batch.astype(jnp.int32), jnp.full((NP - N,), 1 << 20, jnp.int32)]
    ).reshape(NP, 1)
    zrow = jnp.zeros((ROWS_PER_TILE, D), f32)
    onesrow = jnp.ones((CK, D), f32)
    b0r, b1r, b2r, b3r = (v.reshape(1, D) for v in (b0, b1, b2, b3))
    gam, bet = gamma.reshape(1, D), beta.reshape(1, D)

    degp = _sc_degree(dst_p, zrow, onesrow)

    dinv, g = pl.pallas_call(
        _t0_body, out_shape=[_sds((NP, 1)), _sds((NP, D))])(degp, x_p, W0)

    parts = _sc_scatter(g, src_p, dst_p, zrow)
    h, g = pl.pallas_call(
        _t1_body, out_shape=[_sds((NP, D)), _sds((NP, D))])(
            parts, g, dinv, b0r, W1)

    for bb, wn in ((b1r, W2), (b2r, W3)):
        parts = _sc_scatter(g, src_p, dst_p, zrow)
        h, g = pl.pallas_call(
            _tmid_body, out_shape=[_sds((NP, D)), _sds((NP, D))])(
                parts, g, h, dinv, bb, gam, bet, wn)

    parts = _sc_scatter(g, src_p, dst_p, zrow)
    y = pl.pallas_call(_tlast_body, out_shape=_sds((128, 10)))(
        parts, g, h, dinv, b3r, gam, bet, batch_p, Wm0, bm0.reshape(1, -1),
        Wm1, bm1.reshape(1, -1), Wm2, bm2.reshape(1, -1))
    return y
